# baseline (speedup 1.0000x reference)
import functools

import jax
import jax.numpy as jnp
from jax.experimental import pallas as pl
from jax.experimental.pallas import tpu as pltpu


def _body(x_ref, adj_ref, hm_ref, spr_ref, tspr_ref, par_ref, wi_ref,
          wri_ref, wa_ref, wra_ref, bri_ref, bra_ref, allhf_ref, he_ref,
          ind_ref, *, HD, E):
    xb = x_ref[0].astype(jnp.bfloat16)
    v_i = jnp.dot(xb, wi_ref[...].astype(jnp.bfloat16),
                  preferred_element_type=jnp.float32)
    v_a = jnp.dot(xb, wa_ref[...].astype(jnp.bfloat16),
                  preferred_element_type=jnp.float32)
    res_i = jnp.dot(xb, wri_ref[...].astype(jnp.bfloat16),
                    preferred_element_type=jnp.float32) + bri_ref[...]
    res_a = jnp.dot(xb, wra_ref[...].astype(jnp.bfloat16),
                    preferred_element_type=jnp.float32) + bra_ref[...]

    a1_i = par_ref[0:1, :]
    a2_i = par_ref[1:2, :]
    a1_a = par_ref[3:4, :]
    a2_a = par_ref[4:5, :]

    def weights(v, a1, a2):
        f2 = jax.lax.dot_general(a2, v, (((1,), (1,)), ((), ())),
                                 preferred_element_type=jnp.float32)
        m2 = jnp.max(f2)
        t = f2 - m2
        wp_c = jnp.exp(t).astype(jnp.bfloat16)
        wn_c = jnp.exp(0.2 * t).astype(jnp.bfloat16)
        nf2 = (-f2).astype(jnp.bfloat16)
        f1 = jax.lax.dot_general(v, a1, (((1,), (1,)), ((), ())),
                                 preferred_element_type=jnp.float32)
        u = f1 + m2
        m = jnp.where(u > 0, u, 0.2 * u)
        wp_r = jnp.exp(u - m).astype(jnp.bfloat16)
        wn_r = jnp.exp(0.2 * u - m).astype(jnp.bfloat16)
        f1b = f1.astype(jnp.bfloat16)
        return jnp.where(f1b > nf2, wp_r * wp_c, wn_r * wn_c)

    hmb = hm_ref[...].astype(jnp.bfloat16)
    sprb = spr_ref[...].astype(jnp.bfloat16)
    mb = jnp.dot(hmb, sprb,
                 preferred_element_type=jnp.float32)
    mbb = mb.astype(jnp.bfloat16)
    vib = v_i.astype(jnp.bfloat16)
    T = jnp.concatenate(
        [mbb[:, e * HD:(e + 1) * HD] * vib for e in range(E)] + [hmb],
        axis=1)

    w = weights(v_i, a1_i, a2_i)
    acc = jnp.dot(w, T, preferred_element_type=jnp.float32)
    rden = 1.0 / acc[:, E * HD:E * HD + E]
    rdenb = jnp.dot(rden, spr_ref[...],
                    preferred_element_type=jnp.float32)

    res_t = jnp.dot(res_i, tspr_ref[...],
                    preferred_element_type=jnp.float32)
    z = acc[:, :E * HD] * rdenb + res_t
    after = jnp.where(z > 0, z, jnp.exp(z) - 1.0)
    masked = mb * after
    he_row = jnp.max(masked + 2.0 * mb, axis=0, keepdims=True) - 2.0
    for e in range(E):
        sl = slice(e * HD, (e + 1) * HD)
        allhf_ref[e, 0] = masked[:, sl]
    he_ref[0] = jnp.concatenate(
        [he_row[:, e * HD:(e + 1) * HD] for e in range(E)], axis=0)

    w_a = weights(v_a, a1_a, a2_a) * adj_ref[...]
    vab = jnp.concatenate(
        [v_a.astype(jnp.bfloat16),
         jnp.ones((v_a.shape[0], 1), jnp.bfloat16),
         jnp.zeros((v_a.shape[0], 7), jnp.bfloat16)], axis=1)
    acc_a = jnp.dot(w_a, vab, preferred_element_type=jnp.float32)
    rden_a = 1.0 / acc_a[:, HD:HD + 1]
    ones_row = jnp.ones((1, HD), jnp.float32)
    rdab = jnp.dot(rden_a, ones_row, preferred_element_type=jnp.float32)
    z = acc_a[:, 0:HD] * rdab + res_a
    ind_ref[0] = jnp.where(z > 0, z, jnp.exp(z) - 1.0)


def kernel(x, H, adj, nhid, W_i, a1_i, a2_i, b_i, Wres_i, bres_i,
           W_a, a1_a, a2_a, b_a, Wres_a, bres_a):
    B, S, F = x.shape
    HD = W_i.shape[1]
    E = H.shape[1]

    hm = (H != 0).astype(jnp.float32)
    adjb = adj.astype(jnp.bfloat16)
    zeros = jnp.zeros((HD,), jnp.float32)
    params = jnp.stack([a1_i[:, 0], a2_i[:, 0], zeros,
                        a1_a[:, 0], a2_a[:, 0], zeros, zeros, zeros])
    bri = (bres_i + b_i)[None, :]
    bra = (bres_a + b_a)[None, :]
    spread = (jnp.arange(E)[:, None] ==
              (jnp.arange(E * HD) // HD)[None, :]).astype(jnp.float32)
    tspread = (jnp.arange(HD)[:, None] ==
               (jnp.arange(E * HD) % HD)[None, :]).astype(jnp.float32)

    allhf, he, ind = pl.pallas_call(
        functools.partial(_body, HD=HD, E=E),
        grid=(B,),
        in_specs=[
            pl.BlockSpec((1, S, F), lambda b: (b, 0, 0)),
            pl.BlockSpec((S, S), lambda b: (0, 0)),
            pl.BlockSpec((S, E), lambda b: (0, 0)),
            pl.BlockSpec((E, E * HD), lambda b: (0, 0)),
            pl.BlockSpec((HD, E * HD), lambda b: (0, 0)),
            pl.BlockSpec((8, HD), lambda b: (0, 0)),
            pl.BlockSpec((F, HD), lambda b: (0, 0)),
            pl.BlockSpec((F, HD), lambda b: (0, 0)),
            pl.BlockSpec((F, HD), lambda b: (0, 0)),
            pl.BlockSpec((F, HD), lambda b: (0, 0)),
            pl.BlockSpec((1, HD), lambda b: (0, 0)),
            pl.BlockSpec((1, HD), lambda b: (0, 0)),
        ],
        out_specs=[
            pl.BlockSpec((E, 1, S, HD), lambda b: (0, b, 0, 0)),
            pl.BlockSpec((1, E, HD), lambda b: (b, 0, 0)),
            pl.BlockSpec((1, S, HD), lambda b: (b, 0, 0)),
        ],
        out_shape=[
            jax.ShapeDtypeStruct((E, B, S, HD), jnp.float32),
            jax.ShapeDtypeStruct((B, E, HD), jnp.float32),
            jax.ShapeDtypeStruct((B, S, HD), jnp.float32),
        ],
        compiler_params=pltpu.CompilerParams(
            vmem_limit_bytes=100 * 1024 * 1024),
    )(x, adjb, hm, spread, tspread, params, W_i, Wres_i, W_a, Wres_a,
      bri, bra)

    return (allhf, he, ind)


# speedup vs baseline: 1.0632x; 1.0632x over previous
import functools

import jax
import jax.numpy as jnp
from jax.experimental import pallas as pl
from jax.experimental.pallas import tpu as pltpu


def _body(x_ref, adj_ref, hm_ref, spr_ref, tspr_ref, par_ref, wi_ref,
          wri_ref, wa_ref, wra_ref, bri_ref, bra_ref, allhf_ref, he_ref,
          ind_ref, *, HD, E):
    xb = x_ref[0].astype(jnp.bfloat16)
    v_i = jnp.dot(xb, wi_ref[...].astype(jnp.bfloat16),
                  preferred_element_type=jnp.float32)
    v_a = jnp.dot(xb, wa_ref[...].astype(jnp.bfloat16),
                  preferred_element_type=jnp.float32)
    res_i = jnp.dot(xb, wri_ref[...].astype(jnp.bfloat16),
                    preferred_element_type=jnp.float32) + bri_ref[...]
    res_a = jnp.dot(xb, wra_ref[...].astype(jnp.bfloat16),
                    preferred_element_type=jnp.float32) + bra_ref[...]

    a1_i = par_ref[0:1, :]
    a2_i = par_ref[1:2, :]
    a1_a = par_ref[3:4, :]
    a2_a = par_ref[4:5, :]

    def weights(v, a1, a2):
        f2 = jax.lax.dot_general(a2, v, (((1,), (1,)), ((), ())),
                                 preferred_element_type=jnp.float32)
        m2 = jnp.max(f2)
        t = f2 - m2
        wp_c = jnp.exp(t).astype(jnp.bfloat16)
        wn_c = jnp.exp(0.2 * t).astype(jnp.bfloat16)
        nf2 = (-f2).astype(jnp.bfloat16)
        f1 = jax.lax.dot_general(v, a1, (((1,), (1,)), ((), ())),
                                 preferred_element_type=jnp.float32)
        u = f1 + m2
        m = jnp.where(u > 0, u, 0.2 * u)
        wp_r = jnp.exp(u - m).astype(jnp.bfloat16)
        wn_r = jnp.exp(0.2 * u - m).astype(jnp.bfloat16)
        f1b = f1.astype(jnp.bfloat16)
        return jnp.where(f1b > nf2, wp_r * wp_c, wn_r * wn_c)

    hmb = hm_ref[...].astype(jnp.bfloat16)
    sprb = spr_ref[...].astype(jnp.bfloat16)
    mb = jnp.dot(hmb, sprb,
                 preferred_element_type=jnp.float32)
    mbb = mb.astype(jnp.bfloat16)
    vib = v_i.astype(jnp.bfloat16)
    T = jnp.concatenate(
        [mbb[:, e * HD:(e + 1) * HD] * vib for e in range(E)] + [hmb],
        axis=1)

    w = weights(v_i, a1_i, a2_i)
    acc = jnp.dot(w, T, preferred_element_type=jnp.float32)
    rden = 1.0 / acc[:, E * HD:E * HD + E]
    rdenb = jnp.dot(rden, spr_ref[...],
                    preferred_element_type=jnp.float32)

    he_parts = []
    for e in range(E):
        sl = slice(e * HD, (e + 1) * HD)
        z = acc[:, sl] * rdenb[:, sl] + res_i
        after = jnp.where(z > 0, z, jnp.exp(z) - 1.0)
        masked = mb[:, sl] * after
        allhf_ref[e, 0] = masked
        he_parts.append(jnp.max(masked + 2.0 * mb[:, sl],
                                axis=0, keepdims=True))
    he_ref[0] = jnp.concatenate(he_parts, axis=0) - 2.0

    w_a = weights(v_a, a1_a, a2_a) * adj_ref[...].astype(jnp.bfloat16)
    vab = jnp.concatenate(
        [v_a.astype(jnp.bfloat16),
         jnp.ones((v_a.shape[0], 1), jnp.bfloat16),
         jnp.zeros((v_a.shape[0], 7), jnp.bfloat16)], axis=1)
    acc_a = jnp.dot(w_a, vab, preferred_element_type=jnp.float32)
    rden_a = 1.0 / acc_a[:, HD:HD + 1]
    ones_row = jnp.ones((1, HD), jnp.float32)
    rdab = jnp.dot(rden_a, ones_row, preferred_element_type=jnp.float32)
    z = acc_a[:, 0:HD] * rdab + res_a
    ind_ref[0] = jnp.where(z > 0, z, jnp.exp(z) - 1.0)


def kernel(x, H, adj, nhid, W_i, a1_i, a2_i, b_i, Wres_i, bres_i,
           W_a, a1_a, a2_a, b_a, Wres_a, bres_a):
    B, S, F = x.shape
    HD = W_i.shape[1]
    E = H.shape[1]

    hm = (H != 0).astype(jnp.float32)
    zeros = jnp.zeros((HD,), jnp.float32)
    params = jnp.stack([a1_i[:, 0], a2_i[:, 0], zeros,
                        a1_a[:, 0], a2_a[:, 0], zeros, zeros, zeros])
    bri = (bres_i + b_i)[None, :]
    bra = (bres_a + b_a)[None, :]
    spread = (jnp.arange(E)[:, None] ==
              (jnp.arange(E * HD) // HD)[None, :]).astype(jnp.float32)
    tspread = (jnp.arange(HD)[:, None] ==
               (jnp.arange(E * HD) % HD)[None, :]).astype(jnp.float32)

    allhf, he, ind = pl.pallas_call(
        functools.partial(_body, HD=HD, E=E),
        grid=(B,),
        in_specs=[
            pl.BlockSpec((1, S, F), lambda b: (b, 0, 0)),
            pl.BlockSpec((S, S), lambda b: (0, 0)),
            pl.BlockSpec((S, E), lambda b: (0, 0)),
            pl.BlockSpec((E, E * HD), lambda b: (0, 0)),
            pl.BlockSpec((HD, E * HD), lambda b: (0, 0)),
            pl.BlockSpec((8, HD), lambda b: (0, 0)),
            pl.BlockSpec((F, HD), lambda b: (0, 0)),
            pl.BlockSpec((F, HD), lambda b: (0, 0)),
            pl.BlockSpec((F, HD), lambda b: (0, 0)),
            pl.BlockSpec((F, HD), lambda b: (0, 0)),
            pl.BlockSpec((1, HD), lambda b: (0, 0)),
            pl.BlockSpec((1, HD), lambda b: (0, 0)),
        ],
        out_specs=[
            pl.BlockSpec((E, 1, S, HD), lambda b: (0, b, 0, 0)),
            pl.BlockSpec((1, E, HD), lambda b: (b, 0, 0)),
            pl.BlockSpec((1, S, HD), lambda b: (b, 0, 0)),
        ],
        out_shape=[
            jax.ShapeDtypeStruct((E, B, S, HD), jnp.float32),
            jax.ShapeDtypeStruct((B, E, HD), jnp.float32),
            jax.ShapeDtypeStruct((B, S, HD), jnp.float32),
        ],
        compiler_params=pltpu.CompilerParams(
            vmem_limit_bytes=100 * 1024 * 1024),
    )(x, adj, hm, spread, tspread, params, W_i, Wres_i, W_a, Wres_a,
      bri, bra)

    return (allhf, he, ind)


# speedup vs baseline: 1.0705x; 1.0069x over previous
import functools

import jax
import jax.numpy as jnp
from jax.experimental import pallas as pl
from jax.experimental.pallas import tpu as pltpu


def _body(x_ref, adj_ref, hm_ref, spr_ref, par_ref, wi_ref,
          wri_ref, wa_ref, wra_ref, bri_ref, bra_ref, allhf_ref, he_ref,
          ind_ref, *, HD, E):
    xb = x_ref[0].astype(jnp.bfloat16)
    v_i = jnp.dot(xb, wi_ref[...].astype(jnp.bfloat16),
                  preferred_element_type=jnp.float32)
    v_a = jnp.dot(xb, wa_ref[...].astype(jnp.bfloat16),
                  preferred_element_type=jnp.float32)
    res_i = jnp.dot(xb, wri_ref[...].astype(jnp.bfloat16),
                    preferred_element_type=jnp.float32) + bri_ref[...]
    res_a = jnp.dot(xb, wra_ref[...].astype(jnp.bfloat16),
                    preferred_element_type=jnp.float32) + bra_ref[...]

    a1_i = par_ref[0:1, :]
    a2_i = par_ref[1:2, :]
    a1_a = par_ref[3:4, :]
    a2_a = par_ref[4:5, :]

    def weights(v, a1, a2):
        f2 = jax.lax.dot_general(a2, v, (((1,), (1,)), ((), ())),
                                 preferred_element_type=jnp.float32)
        m2 = jnp.max(f2)
        t = f2 - m2
        wp_c = jnp.exp(t).astype(jnp.bfloat16)
        wn_c = jnp.exp(0.2 * t).astype(jnp.bfloat16)
        nf2 = (-f2).astype(jnp.bfloat16)
        f1 = jax.lax.dot_general(v, a1, (((1,), (1,)), ((), ())),
                                 preferred_element_type=jnp.float32)
        u = f1 + m2
        m = jnp.where(u > 0, u, 0.2 * u)
        wp_r = jnp.exp(u - m).astype(jnp.bfloat16)
        wn_r = jnp.exp(0.2 * u - m).astype(jnp.bfloat16)
        f1b = f1.astype(jnp.bfloat16)
        return jnp.where(f1b > nf2, wp_r * wp_c, wn_r * wn_c)

    hmb = hm_ref[...].astype(jnp.bfloat16)
    sprb = spr_ref[...].astype(jnp.bfloat16)
    mb = jnp.dot(hmb, sprb,
                 preferred_element_type=jnp.float32)
    mbb = mb.astype(jnp.bfloat16)
    vib = v_i.astype(jnp.bfloat16)
    T = jnp.concatenate(
        [mbb[:, e * HD:(e + 1) * HD] * vib for e in range(E)] + [hmb],
        axis=1)

    w = weights(v_i, a1_i, a2_i)
    acc = jnp.dot(w, T, preferred_element_type=jnp.float32)
    rden = 1.0 / acc[:, E * HD:E * HD + E]
    rdenb = jnp.dot(rden, spr_ref[...],
                    preferred_element_type=jnp.float32)

    he_parts = []
    for e in range(E):
        sl = slice(e * HD, (e + 1) * HD)
        z = acc[:, sl] * rdenb[:, sl] + res_i
        after = jnp.where(z > 0, z, jnp.exp(z) - 1.0)
        masked = mb[:, sl] * after
        allhf_ref[e, 0] = masked
        he_parts.append(jnp.max(masked + 2.0 * mb[:, sl],
                                axis=0, keepdims=True))
    he_ref[0] = jnp.concatenate(he_parts, axis=0) - 2.0

    w_a = weights(v_a, a1_a, a2_a) * adj_ref[...].astype(jnp.bfloat16)
    vab = jnp.concatenate(
        [v_a.astype(jnp.bfloat16),
         jnp.ones((v_a.shape[0], 1), jnp.bfloat16),
         jnp.zeros((v_a.shape[0], 7), jnp.bfloat16)], axis=1)
    acc_a = jnp.dot(w_a, vab, preferred_element_type=jnp.float32)
    rden_a = 1.0 / acc_a[:, HD:HD + 1]
    ones_row = jnp.ones((1, HD), jnp.float32)
    rdab = jnp.dot(rden_a, ones_row, preferred_element_type=jnp.float32)
    z = acc_a[:, 0:HD] * rdab + res_a
    ind_ref[0] = jnp.where(z > 0, z, jnp.exp(z) - 1.0)


def kernel(x, H, adj, nhid, W_i, a1_i, a2_i, b_i, Wres_i, bres_i,
           W_a, a1_a, a2_a, b_a, Wres_a, bres_a):
    B, S, F = x.shape
    HD = W_i.shape[1]
    E = H.shape[1]

    hm = (H != 0).astype(jnp.float32)
    zeros = jnp.zeros((HD,), jnp.float32)
    params = jnp.stack([a1_i[:, 0], a2_i[:, 0], zeros,
                        a1_a[:, 0], a2_a[:, 0], zeros, zeros, zeros])
    bri = (bres_i + b_i)[None, :]
    bra = (bres_a + b_a)[None, :]
    spread = (jnp.arange(E)[:, None] ==
              (jnp.arange(E * HD) // HD)[None, :]).astype(jnp.float32)

    allhf, he, ind = pl.pallas_call(
        functools.partial(_body, HD=HD, E=E),
        grid=(B,),
        in_specs=[
            pl.BlockSpec((1, S, F), lambda b: (b, 0, 0)),
            pl.BlockSpec((S, S), lambda b: (0, 0)),
            pl.BlockSpec((S, E), lambda b: (0, 0)),
            pl.BlockSpec((E, E * HD), lambda b: (0, 0)),
            pl.BlockSpec((8, HD), lambda b: (0, 0)),
            pl.BlockSpec((F, HD), lambda b: (0, 0)),
            pl.BlockSpec((F, HD), lambda b: (0, 0)),
            pl.BlockSpec((F, HD), lambda b: (0, 0)),
            pl.BlockSpec((F, HD), lambda b: (0, 0)),
            pl.BlockSpec((1, HD), lambda b: (0, 0)),
            pl.BlockSpec((1, HD), lambda b: (0, 0)),
        ],
        out_specs=[
            pl.BlockSpec((E, 1, S, HD), lambda b: (0, b, 0, 0)),
            pl.BlockSpec((1, E, HD), lambda b: (b, 0, 0)),
            pl.BlockSpec((1, S, HD), lambda b: (b, 0, 0)),
        ],
        out_shape=[
            jax.ShapeDtypeStruct((E, B, S, HD), jnp.float32),
            jax.ShapeDtypeStruct((B, E, HD), jnp.float32),
            jax.ShapeDtypeStruct((B, S, HD), jnp.float32),
        ],
        compiler_params=pltpu.CompilerParams(
            vmem_limit_bytes=100 * 1024 * 1024),
    )(x, adj, hm, spread, params, W_i, Wres_i, W_a, Wres_a, bri, bra)

    return (allhf, he, ind)
